# baseline (speedup 1.0000x reference)
import jax
import jax.numpy as jnp
from jax.experimental import pallas as pl


def _gcn_kernel(a1_ref, a2_ref, x_ref, w_ref, b_ref, z_ref):
    h = a1_ref.shape[-1]
    bm = a1_ref.shape[0]
    a1 = a1_ref[...].reshape(bm, h)
    a2 = a2_ref[...].reshape(bm, h)
    t = jnp.dot(a1, x_ref[pl.ds(0, h), :], preferred_element_type=jnp.float32)
    t += jnp.dot(a2, x_ref[pl.ds(h, h), :], preferred_element_type=jnp.float32)
    z_ref[...] = jnp.dot(t, w_ref[...],
                         preferred_element_type=jnp.float32) + b_ref[...]


@jax.jit
def kernel(X, A_hat, W, b):
    n, d_in = X.shape
    d_out = W.shape[1]
    b2 = b.reshape(1, d_out)

    bm = 256
    h = n // 2
    A4 = A_hat.reshape(n, 2, 1, h)
    grid = (pl.cdiv(n, bm),)
    Z = pl.pallas_call(
        _gcn_kernel,
        grid=grid,
        in_specs=[
            pl.BlockSpec((bm, 1, 1, h), lambda i: (i, 0, 0, 0)),
            pl.BlockSpec((bm, 1, 1, h), lambda i: (i, 1, 0, 0)),
            pl.BlockSpec((n, d_in), lambda i: (0, 0)),
            pl.BlockSpec((d_in, d_out), lambda i: (0, 0)),
            pl.BlockSpec((1, d_out), lambda i: (0, 0)),
        ],
        out_specs=pl.BlockSpec((bm, d_out), lambda i: (i, 0)),
        out_shape=jax.ShapeDtypeStruct((n, d_out), jnp.float32),
    )(A4, A4, X, W, b2)
    return Z


# speedup vs baseline: 22.2027x; 22.2027x over previous
import jax
import jax.numpy as jnp
from jax.experimental import pallas as pl


def _gcn_kernel(a1_ref, a2_ref, x_ref, w_ref, b_ref, z_ref):
    bm = a1_ref.shape[0]
    x = x_ref[...]
    w = w_ref[...]
    t1 = jnp.dot(a1_ref[...], x, preferred_element_type=jnp.float32)
    z_ref[pl.ds(0, bm), :] = jnp.dot(
        t1, w, preferred_element_type=jnp.float32) + b_ref[...]
    t2 = jnp.dot(a2_ref[...], x, preferred_element_type=jnp.float32)
    z_ref[pl.ds(bm, bm), :] = jnp.dot(
        t2, w, preferred_element_type=jnp.float32) + b_ref[...]


@jax.jit
def kernel(X, A_hat, W, b):
    n, d_in = X.shape
    d_out = W.shape[1]
    b2 = b.reshape(1, d_out)

    bm = 256
    grid = (pl.cdiv(n, 2 * bm),)
    Z = pl.pallas_call(
        _gcn_kernel,
        grid=grid,
        in_specs=[
            pl.BlockSpec((bm, n), lambda i: (2 * i, 0)),
            pl.BlockSpec((bm, n), lambda i: (2 * i + 1, 0)),
            pl.BlockSpec((n, d_in), lambda i: (0, 0)),
            pl.BlockSpec((d_in, d_out), lambda i: (0, 0)),
            pl.BlockSpec((1, d_out), lambda i: (0, 0)),
        ],
        out_specs=pl.BlockSpec((2 * bm, d_out), lambda i: (i, 0)),
        out_shape=jax.ShapeDtypeStruct((n, d_out), jnp.float32),
    )(A_hat, A_hat, X, W, b2)
    return Z


# speedup vs baseline: 24.1595x; 1.0881x over previous
import jax
import jax.numpy as jnp
from jax.experimental import pallas as pl


def _gcn_kernel(a_ref, x_ref, w_ref, b_ref, z_ref):
    t = jnp.dot(a_ref[...], x_ref[...], preferred_element_type=jnp.float32)
    z_ref[...] = jnp.dot(t, w_ref[...],
                         preferred_element_type=jnp.float32) + b_ref[...]


@jax.jit
def kernel(X, A_hat, W, b):
    n, d_in = X.shape
    d_out = W.shape[1]
    b2 = b.reshape(1, d_out)

    bm = 400
    grid = (pl.cdiv(n, bm),)
    Z = pl.pallas_call(
        _gcn_kernel,
        grid=grid,
        in_specs=[
            pl.BlockSpec((bm, n), lambda i: (i, 0)),
            pl.BlockSpec((n, d_in), lambda i: (0, 0)),
            pl.BlockSpec((d_in, d_out), lambda i: (0, 0)),
            pl.BlockSpec((1, d_out), lambda i: (0, 0)),
        ],
        out_specs=pl.BlockSpec((bm, d_out), lambda i: (i, 0)),
        out_shape=jax.ShapeDtypeStruct((n, d_out), jnp.float32),
    )(A_hat, X, W, b2)
    return Z
